# baseline (speedup 1.0000x reference)
import functools

import jax
import jax.numpy as jnp
from jax import lax
from jax.experimental import pallas as pl
from jax.experimental.pallas import tpu as pltpu
from jax.experimental.pallas import tpu_sc as plsc

NC = 2
NS = 16
NW = NC * NS
CH = 64


def _sc_scatter_call(d, nhalf, n_pad):
    rpz = n_pad // NS
    zfull = rpz // CH
    zrem = rpz % CH

    mesh = plsc.VectorSubcoreMesh(
        core_axis_name="c", subcore_axis_name="s", num_cores=NC,
        num_subcores=NS)

    @functools.partial(
        pl.kernel,
        mesh=mesh,
        out_type=jax.ShapeDtypeStruct((NC, n_pad, d), jnp.float32),
        scratch_types=[
            pltpu.VMEM((nhalf + 1, CH), jnp.int32),
            pltpu.VMEM((nhalf, CH), jnp.int32),
            pltpu.VMEM((CH, d), jnp.float32),
            pltpu.VMEM((CH, d), jnp.float32),
            pltpu.VMEM_SHARED((n_pad, d), jnp.float32),
            pltpu.SemaphoreType.DMA,
            pltpu.SemaphoreType.DMA,
        ],
    )
    def scatter_kernel(rows_hbm, src_hbm, dst_hbm, out_hbm,
                       src_v, dst_v, buf0, buf1, acc_sh, sem0, sem1):
        c = lax.axis_index("c")
        s = lax.axis_index("s")
        wid = s * NC + c

        zero16 = jnp.zeros((16,), jnp.float32)

        def zrow(i, carry):
            for j in range(d // 16):
                buf0[i, pl.ds(j * 16, 16)] = zero16
            return carry

        lax.fori_loop(0, CH, zrow, 0)
        for k in range(zfull):
            pltpu.sync_copy(buf0, acc_sh.at[pl.ds(s * rpz + k * CH, CH)])
        if zrem:
            pltpu.sync_copy(
                buf0.at[pl.ds(0, zrem)],
                acc_sh.at[pl.ds(s * rpz + zfull * CH, zrem)])
        plsc.subcore_barrier()

        def run_half(h):
            pltpu.sync_copy(src_hbm.at[wid, h], src_v)
            pltpu.sync_copy(dst_hbm.at[wid, h], dst_v)
            pltpu.async_copy(rows_hbm.at[src_v.at[0]], buf0, sem0)

            def step(i, carry):
                j0 = 2 * i
                pltpu.async_copy(rows_hbm.at[src_v.at[j0 + 1]], buf1, sem1)
                pltpu.make_async_copy(
                    rows_hbm.at[src_v.at[j0]], buf0, sem0).wait()
                pltpu.sync_copy(buf0, acc_sh.at[dst_v.at[j0]], add=True)
                pltpu.async_copy(rows_hbm.at[src_v.at[j0 + 2]], buf0, sem0)
                pltpu.make_async_copy(
                    rows_hbm.at[src_v.at[j0 + 1]], buf1, sem1).wait()
                pltpu.sync_copy(buf1, acc_sh.at[dst_v.at[j0 + 1]], add=True)
                return carry

            lax.fori_loop(0, nhalf // 2, step, 0)
            pltpu.make_async_copy(rows_hbm.at[src_v.at[0]], buf0, sem0).wait()

        run_half(0)
        run_half(1)
        plsc.subcore_barrier()

        pltpu.sync_copy(acc_sh.at[pl.ds(s * rpz, rpz)],
                        out_hbm.at[c, pl.ds(s * rpz, rpz)])

    return scatter_kernel


def _combine_matmul_relu_call(p, w, rows_blk):
    _, n, d = p.shape

    def body(p_ref, w_ref, o_ref):
        agg = p_ref[0] + p_ref[1]
        o_ref[...] = jnp.maximum(
            jnp.dot(agg, w_ref[...], preferred_element_type=jnp.float32), 0.0)

    return pl.pallas_call(
        body,
        grid=(n // rows_blk,),
        in_specs=[
            pl.BlockSpec((NC, rows_blk, d), lambda i: (0, i, 0)),
            pl.BlockSpec((d, d), lambda i: (0, 0)),
        ],
        out_specs=pl.BlockSpec((rows_blk, d), lambda i: (i, 0)),
        out_shape=jax.ShapeDtypeStruct((n, d), jnp.float32),
    )(p, w)


def kernel(A, X, W1, W2):
    x = X[0]
    n, d = x.shape
    e = A.shape[1]

    epw = -(-e // (NW * 4 * CH)) * 4 * CH
    nhalf = epw // (2 * CH)
    e_pad = NW * epw
    n_pad = -(-(n + 1) // 128) * 128

    trash = n + jnp.arange(e_pad - e, dtype=jnp.int32) % (n_pad - n)
    src = jnp.concatenate(
        [A[0], jnp.zeros((e_pad - e,), jnp.int32)]).reshape(NW, 2, nhalf, CH)
    src = jnp.concatenate(
        [src, jnp.zeros((NW, 2, 1, CH), jnp.int32)], axis=2)
    dst = jnp.concatenate([A[1], trash]).reshape(NW, 2, nhalf, CH)

    scatter = _sc_scatter_call(d, nhalf, n_pad)

    blk = n_pad // 8
    p1 = scatter(x, src, dst)
    h1 = _combine_matmul_relu_call(p1, W1, blk)
    p2 = scatter(h1, src, dst)
    out = _combine_matmul_relu_call(p2, W2, blk)
    return out[None, :n, :]
